# baseline (speedup 1.0000x reference)
import jax
import jax.numpy as jnp
from jax.experimental import pallas as pl

_K = 16
_R = 256
_H = 16

_DN = (((1,), (0,)), ((), ()))


def _sort16(u):
    u = list(u)
    k = 2
    while k <= 16:
        j = k // 2
        while j >= 1:
            for i in range(16):
                p = i ^ j
                if p > i:
                    lo = jnp.minimum(u[i], u[p])
                    hi = jnp.maximum(u[i], u[p])
                    if (i & k) == 0:
                        u[i], u[p] = lo, hi
                    else:
                        u[i], u[p] = hi, lo
            j //= 2
        k *= 2
    return u


def _resort16(u):
    u = list(u)
    j = 8
    while j >= 1:
        for i in range(16):
            p = i ^ j
            if p > i:
                lo = jnp.minimum(u[i], u[p])
                hi = jnp.maximum(u[i], u[p])
                u[i], u[p] = lo, hi
        j //= 2
    return u


def _merge_lo(a, b):
    c = [jnp.minimum(a[i], b[15 - i]) for i in range(16)]
    return _resort16(c)


def _top16_rows(d2, M):
    units = [d2[i * _H:(i + 1) * _H, :] for i in range(M // _H)]
    runs = [_sort16(units[16 * r:16 * (r + 1)])
            for r in range(len(units) // 16)]
    while len(runs) > 1:
        runs = [_merge_lo(runs[2 * i], runs[2 * i + 1])
                for i in range(len(runs) // 2)]
    return jnp.concatenate(runs[0], axis=0)


def _voroloss_block(sp_ref, spT_ref, pT_ref, out_ref):
    S = sp_ref[...]
    ST = spT_ref[...]
    P = pT_ref[...]
    M = S.shape[0]
    R = P.shape[1]

    sx = S[:, 0:1]
    sy = S[:, 1:2]
    sz = S[:, 2:3]
    px = P[0:1, :]
    py = P[1:2, :]
    pz = P[2:3, :]

    f32 = jnp.float32
    bf = jnp.bfloat16
    s2 = (sx * sx + sy * sy) + sz * sz
    p2 = (px * px + py * py) + pz * pz

    ps = jax.lax.dot_general(S.astype(bf), P.astype(bf), _DN,
                             preferred_element_type=f32)
    d2 = (p2 + s2) - 2.0 * ps

    inf = f32(jnp.inf)
    cand = _top16_rows(d2, M)
    m0 = jnp.min(cand, axis=0, keepdims=True)
    m = m0
    for _ in range(_K - 1):
        m = jnp.min(jnp.where(cand > m, cand, inf),
                    axis=0, keepdims=True)
    T = m

    ic = jax.lax.broadcasted_iota(jnp.int32, (M, 1), 0)
    i0 = jnp.min(jnp.where(d2 == m0, ic, M), axis=0, keepdims=True)
    sel0f = (ic == i0).astype(f32)
    hi = jax.lax.Precision.HIGHEST
    S0 = jax.lax.dot_general(ST, sel0f, _DN, precision=hi,
                             preferred_element_type=f32)
    s0x = S0[0:1, :]
    s0y = S0[1:2, :]
    s0z = S0[2:3, :]

    G = jax.lax.dot_general(S, P, _DN, precision=hi,
                            preferred_element_type=f32)
    E = jax.lax.dot_general(S, S0, _DN, precision=hi,
                            preferred_element_type=f32)
    s02 = (s0x * s0x + s0y * s0y) + s0z * s0z
    g0 = s02 - 2.0 * ((px * s0x + py * s0y) + pz * s0z)
    CH = 512
    acc = None
    for c in range(0, M, CH):
        s2c = s2[c:c + CH, :]
        gc = s2c - 2.0 * G[c:c + CH, :]
        el2c = (s2c + s02) - 2.0 * E[c:c + CH, :]
        diffc = gc - g0
        qc = (diffc * diffc) / el2c
        fmaskc = (d2[c:c + CH, :] <= T) & (ic[c:c + CH, :] != i0)
        part = jnp.min(jnp.where(fmaskc, qc, inf), axis=0, keepdims=True)
        acc = part if acc is None else jnp.minimum(acc, part)
    out_ref[...] = (0.25 * acc)[None]


@jax.jit
def kernel(points, spoints):
    N = points.shape[0]
    M = spoints.shape[0]
    R = _R
    grid = N // R
    pT = points.T
    spT = spoints.T
    out = pl.pallas_call(
        _voroloss_block,
        grid=(grid,),
        in_specs=[
            pl.BlockSpec((M, 3), lambda i: (0, 0)),
            pl.BlockSpec((3, M), lambda i: (0, 0)),
            pl.BlockSpec((3, R), lambda i: (0, i)),
        ],
        out_specs=pl.BlockSpec((1, 1, R), lambda i: (i, 0, 0)),
        out_shape=jax.ShapeDtypeStruct((grid, 1, R), jnp.float32),
    )(spoints, spT, pT)
    return out.reshape(N)
